# baseline (speedup 1.0000x reference)
import jax
import jax.numpy as jnp
from jax.experimental import pallas as pl
from jax.experimental.pallas import tpu as pltpu

B, D, H, E, K, DEPTH, OUT = 2048, 768, 768, 8, 2, 4, 768
NSTEP = 1 + E + DEPTH - 1
LOSS_COEF = 0.01
NEG = -3.0e38


def _dot(a, b):
    return jax.lax.dot_general(a, b, (((1,), (0,)), ((), ())),
                               preferred_element_type=jnp.float32)


def _fused_kernel(x_ref, wg_ref, We1_ref, We2_ref, be1_ref, be2_ref,
                  W1_ref, W2_ref, b1_ref, b2_ref, Wo_ref, bo_ref,
                  out_ref, loss_ref, g_ref, xb_ref, acc_ref, h2s_ref):
    i = pl.program_id(0)

    @pl.when(i == 0)
    def _gating():
        x = x_ref[...]
        xb_ref[...] = x.astype(jnp.bfloat16)
        logits = _dot(x, wg_ref[...])
        m1 = jnp.max(logits, axis=1, keepdims=True)
        l2 = jnp.where(logits == m1, NEG, logits)
        m2 = jnp.max(l2, axis=1, keepdims=True)
        top2 = logits >= m2
        ex = jnp.where(top2, jnp.exp(logits - m1), 0.0)
        den = jnp.sum(ex, axis=1, keepdims=True)
        g = ex / den
        g_ref[...] = g
        imp = jnp.sum(g, axis=0, keepdims=True)
        load = jnp.sum(top2.astype(jnp.float32), axis=0, keepdims=True)

        def cv(v):
            m = jnp.sum(v, axis=1, keepdims=True) / E
            var = jnp.sum((v - m) ** 2, axis=1, keepdims=True) / (E - 1)
            return var / (m * m + 1e-10)

        loss_ref[0, 0] = ((cv(imp) + cv(load)) * LOSS_COEF)[0, 0]
        acc_ref[...] = _dot(g, be2_ref[...])

    @pl.when((i >= 1) & (i <= E))
    def _expert():
        e = i - 1
        xb = xb_ref[...]
        w1 = We1_ref[...].astype(jnp.bfloat16)
        h1 = jnp.tanh(_dot(xb, w1) + be1_ref[e][None, :])
        w2 = We2_ref[...].astype(jnp.bfloat16)
        d2 = _dot(h1.astype(jnp.bfloat16), w2)

        def gate_col(ee):
            lane = jax.lax.broadcasted_iota(jnp.int32, (B, E), 1)
            return jnp.sum(jnp.where(lane == ee, g_ref[...], 0.0), axis=1,
                           keepdims=True)

        @pl.when(e % 2 == 0)
        def _park():
            h2s_ref[...] = d2.astype(jnp.bfloat16)

        @pl.when(e % 2 == 1)
        def _accum():
            acc_ref[...] += (gate_col(e - 1)
                             * h2s_ref[...].astype(jnp.float32)
                             + gate_col(e) * d2)

    @pl.when(i > E)
    def _trunk():
        j = i - E - 1
        cur = acc_ref[...]
        w1 = W1_ref[...].astype(jnp.bfloat16)
        h = jnp.tanh(_dot(cur.astype(jnp.bfloat16), w1) + b1_ref[j][None, :])
        w2 = W2_ref[...].astype(jnp.bfloat16)
        h2 = _dot(h.astype(jnp.bfloat16), w2) + b2_ref[j][None, :]
        new = jnp.tanh(h2 + cur)

        @pl.when(i < NSTEP - 1)
        def _store():
            acc_ref[...] = new

        @pl.when(i == NSTEP - 1)
        def _proj():
            wo = Wo_ref[...].astype(jnp.bfloat16)
            out_ref[...] = _dot(new.astype(jnp.bfloat16), wo) + \
                bo_ref[0][None, :]


def kernel(x, w_gate, We1, be1, We2, be2, Wr1, br1, Wr2, br2, Wo, bo):
    out, loss = pl.pallas_call(
        _fused_kernel,
        grid=(NSTEP,),
        in_specs=[
            pl.BlockSpec((B, D), lambda i: (0, 0)),
            pl.BlockSpec((D, E), lambda i: (0, 0)),
            pl.BlockSpec((None, D, H),
                         lambda i: (jnp.clip(i - 1, 0, E - 1), 0, 0)),
            pl.BlockSpec((None, H, H),
                         lambda i: (jnp.clip(i - 1, 0, E - 1), 0, 0)),
            pl.BlockSpec((E, H), lambda i: (0, 0)),
            pl.BlockSpec((E, H), lambda i: (0, 0)),
            pl.BlockSpec((None, H, H),
                         lambda i: (jnp.clip(i - E - 1, 0, DEPTH - 2), 0, 0)),
            pl.BlockSpec((None, H, H),
                         lambda i: (jnp.clip(i - E - 1, 0, DEPTH - 2), 0, 0)),
            pl.BlockSpec((DEPTH - 1, H), lambda i: (0, 0)),
            pl.BlockSpec((DEPTH - 1, H), lambda i: (0, 0)),
            pl.BlockSpec((H, OUT), lambda i: (0, 0)),
            pl.BlockSpec((1, OUT), lambda i: (0, 0)),
        ],
        out_specs=[
            pl.BlockSpec((B, OUT), lambda i: (0, 0)),
            pl.BlockSpec(memory_space=pltpu.SMEM),
        ],
        out_shape=[
            jax.ShapeDtypeStruct((B, OUT), jnp.float32),
            jax.ShapeDtypeStruct((1, 1), jnp.float32),
        ],
        scratch_shapes=[
            pltpu.VMEM((B, E), jnp.float32),
            pltpu.VMEM((B, D), jnp.bfloat16),
            pltpu.VMEM((B, H), jnp.float32),
            pltpu.VMEM((B, H), jnp.bfloat16),
        ],
    )(x, w_gate, We1, We2, be1, be2, Wr1, Wr2, br1, br2, Wo,
      bo.reshape(1, OUT))

    return out, loss[0, 0]


# speedup vs baseline: 1.0499x; 1.0499x over previous
import jax
import jax.numpy as jnp
from jax.experimental import pallas as pl
from jax.experimental.pallas import tpu as pltpu

B, D, H, E, K, DEPTH, OUT = 2048, 768, 768, 8, 2, 4, 768
NSTEP = 1 + E + DEPTH - 1
LOSS_COEF = 0.01
NEG = -3.0e38


def _dot(a, b):
    return jax.lax.dot_general(a, b, (((1,), (0,)), ((), ())),
                               preferred_element_type=jnp.float32)


def _fused_kernel(x_ref, wg_ref, We1_ref, We2_ref, be1_ref, be2_ref,
                  W1_ref, W2_ref, b1_ref, b2_ref, Wo_ref, bo_ref,
                  out_ref, loss_ref, g_ref, xb_ref, acc_ref):
    i = pl.program_id(0)

    @pl.when(i == 0)
    def _gating():
        x = x_ref[...]
        xb_ref[...] = x.astype(jnp.bfloat16)
        logits = _dot(x, wg_ref[...])
        m1 = jnp.max(logits, axis=1, keepdims=True)
        l2 = jnp.where(logits == m1, NEG, logits)
        m2 = jnp.max(l2, axis=1, keepdims=True)
        top2 = logits >= m2
        ex = jnp.where(top2, jnp.exp(logits - m1), 0.0)
        den = jnp.sum(ex, axis=1, keepdims=True)
        g = ex / den
        g_ref[...] = g
        imp = jnp.sum(g, axis=0, keepdims=True)
        load = jnp.sum(top2.astype(jnp.float32), axis=0, keepdims=True)

        def cv(v):
            m = jnp.sum(v, axis=1, keepdims=True) / E
            var = jnp.sum((v - m) ** 2, axis=1, keepdims=True) / (E - 1)
            return var / (m * m + 1e-10)

        loss_ref[0, 0] = ((cv(imp) + cv(load)) * LOSS_COEF)[0, 0]

    @pl.when((i >= 1) & (i <= E))
    def _expert():
        e = i - 1
        xb = xb_ref[...]
        w1 = We1_ref[...].astype(jnp.bfloat16)
        h1 = jnp.tanh(_dot(xb, w1) + be1_ref[e][None, :])
        w2 = We2_ref[...].astype(jnp.bfloat16)
        h2 = _dot(h1.astype(jnp.bfloat16), w2) + be2_ref[e][None, :]
        lane = jax.lax.broadcasted_iota(jnp.int32, (B, E), 1)
        ge = jnp.sum(jnp.where(lane == e, g_ref[...], 0.0), axis=1,
                     keepdims=True)
        contrib = ge * h2

        @pl.when(e == 0)
        def _init():
            acc_ref[...] = contrib

        @pl.when(e > 0)
        def _accum():
            acc_ref[...] += contrib

    @pl.when(i > E)
    def _trunk():
        j = i - E - 1
        cur = acc_ref[...]
        w1 = W1_ref[...].astype(jnp.bfloat16)
        h = jnp.tanh(_dot(cur.astype(jnp.bfloat16), w1) + b1_ref[j][None, :])
        w2 = W2_ref[...].astype(jnp.bfloat16)
        h2 = _dot(h.astype(jnp.bfloat16), w2) + b2_ref[j][None, :]
        new = jnp.tanh(h2 + cur)

        @pl.when(i < NSTEP - 1)
        def _store():
            acc_ref[...] = new

        @pl.when(i == NSTEP - 1)
        def _proj():
            wo = Wo_ref[...].astype(jnp.bfloat16)
            out_ref[...] = _dot(new.astype(jnp.bfloat16), wo) + \
                bo_ref[0][None, :]


def kernel(x, w_gate, We1, be1, We2, be2, Wr1, br1, Wr2, br2, Wo, bo):
    out, loss = pl.pallas_call(
        _fused_kernel,
        grid=(NSTEP,),
        in_specs=[
            pl.BlockSpec((B, D), lambda i: (0, 0)),
            pl.BlockSpec((D, E), lambda i: (0, 0)),
            pl.BlockSpec((None, D, H),
                         lambda i: (jnp.clip(i - 1, 0, E - 1), 0, 0)),
            pl.BlockSpec((None, H, H),
                         lambda i: (jnp.clip(i - 1, 0, E - 1), 0, 0)),
            pl.BlockSpec((E, H), lambda i: (0, 0)),
            pl.BlockSpec((E, H), lambda i: (0, 0)),
            pl.BlockSpec((None, H, H),
                         lambda i: (jnp.clip(i - E - 1, 0, DEPTH - 2), 0, 0)),
            pl.BlockSpec((None, H, H),
                         lambda i: (jnp.clip(i - E - 1, 0, DEPTH - 2), 0, 0)),
            pl.BlockSpec((DEPTH - 1, H), lambda i: (0, 0)),
            pl.BlockSpec((DEPTH - 1, H), lambda i: (0, 0)),
            pl.BlockSpec((H, OUT), lambda i: (0, 0)),
            pl.BlockSpec((1, OUT), lambda i: (0, 0)),
        ],
        out_specs=[
            pl.BlockSpec((B, OUT), lambda i: (0, 0)),
            pl.BlockSpec(memory_space=pltpu.SMEM),
        ],
        out_shape=[
            jax.ShapeDtypeStruct((B, OUT), jnp.float32),
            jax.ShapeDtypeStruct((1, 1), jnp.float32),
        ],
        scratch_shapes=[
            pltpu.VMEM((B, E), jnp.float32),
            pltpu.VMEM((B, D), jnp.bfloat16),
            pltpu.VMEM((B, H), jnp.float32),
        ],
    )(x, w_gate, We1, We2, be1, be2, Wr1, Wr2, br1, br2, Wo,
      bo.reshape(1, OUT))

    return out, loss[0, 0]
